# Initial kernel scaffold; baseline (speedup 1.0000x reference)
import jax
import jax.numpy as jnp
from jax.experimental import pallas as pl


def kernel(months, hours, month_table, hour_table):
    raise NotImplementedError("write your pallas kernel here")


# speedup vs baseline: 1.6035x; 1.6035x over previous
import dataclasses

import jax
import jax.numpy as jnp
from jax import lax
from jax.experimental import pallas as pl
from jax.experimental.pallas import tpu as pltpu
from jax.experimental.pallas import tpu_sc as plsc

B = 16384
HALF = 64
NC = 2
NS = 16
L = 16
NW = NC * NS
BPW = B // NW
ROWS_PW = 2 * BPW
GCHUNK = 128
NG = ROWS_PW // GCHUNK


def _emb_body(tbl_hbm, months_hbm, hours_hbm, out_hbm, m_v, h_v, idx_v, rows_v, sem):
    wid = lax.axis_index("s") * NC + lax.axis_index("c")
    base = wid * BPW
    pltpu.sync_copy(months_hbm.at[pl.ds(base, BPW)], m_v)
    pltpu.sync_copy(hours_hbm.at[pl.ds(base, BPW)], h_v)
    lane = lax.iota(jnp.int32, L)
    for i in range(BPW // L):
        mm = m_v[pl.ds(i * L, L)]
        hh = h_v[pl.ds(i * L, L)] + 12
        pos = lane * 2 + (2 * i * L)
        plsc.store_scatter(idx_v, [pos], mm)
        plsc.store_scatter(idx_v, [pos + 1], hh)
    copies = [
        pltpu.async_copy(
            tbl_hbm.at[idx_v.at[pl.ds(j * GCHUNK, GCHUNK)]],
            rows_v.at[pl.ds(j * GCHUNK, GCHUNK)],
            sem,
        )
        for j in range(NG)
    ]
    for c in copies:
        c.wait()
    pltpu.sync_copy(rows_v, out_hbm.at[pl.ds(wid * ROWS_PW, ROWS_PW)])


def kernel(months, hours, month_table, hour_table):
    table = jnp.concatenate([month_table, hour_table], axis=0)
    months = months.astype(jnp.int32)
    hours = hours.astype(jnp.int32)
    mesh = plsc.VectorSubcoreMesh(core_axis_name="c", subcore_axis_name="s")
    cp = pltpu.CompilerParams(needs_layout_passes=False, use_tc_tiling_on_sc=False)
    run = pl.kernel(
        _emb_body,
        out_type=jax.ShapeDtypeStruct((2 * B, HALF), jnp.float32),
        mesh=mesh,
        scratch_types=[
            pltpu.VMEM((BPW,), jnp.int32),
            pltpu.VMEM((BPW,), jnp.int32),
            pltpu.VMEM((ROWS_PW,), jnp.int32),
            pltpu.VMEM((ROWS_PW, HALF), jnp.float32),
            pltpu.SemaphoreType.DMA,
        ],
        compiler_params=cp,
    )
    out2 = run(table, months, hours)
    return out2.reshape(B, 2 * HALF)


# speedup vs baseline: 3.3663x; 2.0994x over previous
import jax
import jax.numpy as jnp
from jax import lax
from jax.experimental import pallas as pl
from jax.experimental.pallas import tpu as pltpu
from jax.experimental.pallas import tpu_sc as plsc

B = 16384
D = 128
HALF = 64
NC = 2
NS = 16
L = 16
NW = NC * NS
BPW = B // NW
GROUPS = BPW // L
MT_WORDS = 12 * HALF
HT_WORDS = 24 * HALF
TBL_WORDS = MT_WORDS + HT_WORDS


def _emb_body(mt_hbm, ht_hbm, months_hbm, hours_hbm, out_hbm,
              tbl_v, m_v, h_v, rows_v, sem):
    wid = lax.axis_index("s") * NC + lax.axis_index("c")
    base = wid * BPW
    copies = [
        pltpu.async_copy(mt_hbm, tbl_v.at[pl.ds(0, MT_WORDS)], sem),
        pltpu.async_copy(ht_hbm, tbl_v.at[pl.ds(MT_WORDS, HT_WORDS)], sem),
        pltpu.async_copy(months_hbm.at[pl.ds(base, BPW)], m_v, sem),
        pltpu.async_copy(hours_hbm.at[pl.ds(base, BPW)], h_v, sem),
    ]
    for c in copies:
        c.wait()

    @pl.loop(0, GROUPS)
    def _(g):
        ib = g * L
        mm = m_v[pl.ds(ib, L)] * HALF
        hh = h_v[pl.ds(ib, L)] * HALF + MT_WORDS
        for l in range(L):
            ms = mm[l]
            hs = hh[l]
            dst = (ib + l) * D
            for c in range(0, HALF, L):
                rows_v[pl.ds(dst + c, L)] = tbl_v[pl.ds(ms + c, L)]
                rows_v[pl.ds(dst + HALF + c, L)] = tbl_v[pl.ds(hs + c, L)]

    pltpu.sync_copy(rows_v, out_hbm.at[pl.ds(base * D, BPW * D)])


def kernel(months, hours, month_table, hour_table):
    mesh = plsc.VectorSubcoreMesh(core_axis_name="c", subcore_axis_name="s")
    cp = pltpu.CompilerParams(needs_layout_passes=False, use_tc_tiling_on_sc=False)
    run = pl.kernel(
        _emb_body,
        out_type=jax.ShapeDtypeStruct((B * D,), jnp.float32),
        mesh=mesh,
        scratch_types=[
            pltpu.VMEM((TBL_WORDS,), jnp.float32),
            pltpu.VMEM((BPW,), jnp.int32),
            pltpu.VMEM((BPW,), jnp.int32),
            pltpu.VMEM((BPW * D,), jnp.float32),
            pltpu.SemaphoreType.DMA,
        ],
        compiler_params=cp,
    )
    out = run(month_table.reshape(-1), hour_table.reshape(-1),
              months.astype(jnp.int32), hours.astype(jnp.int32))
    return out.reshape(B, D)


# speedup vs baseline: 3.8743x; 1.1509x over previous
import jax
import jax.numpy as jnp
from jax import lax
from jax.experimental import pallas as pl
from jax.experimental.pallas import tpu as pltpu
from jax.experimental.pallas import tpu_sc as plsc

B = 16384
D = 128
HALF = 64
NC = 2
NS = 16
L = 16
NW = NC * NS
BPW = B // NW
GROUPS = BPW // L
MT_WORDS = 12 * HALF
HT_WORDS = 24 * HALF
TBL_WORDS = MT_WORDS + HT_WORDS


def _emb_body(mt_hbm, ht_hbm, months_hbm, hours_hbm, out_hbm,
              tbl_v, m_v, h_v, rows_v, sem):
    wid = lax.axis_index("s") * NC + lax.axis_index("c")
    base = wid * BPW
    copies = [
        pltpu.async_copy(mt_hbm, tbl_v.at[pl.ds(0, MT_WORDS)], sem),
        pltpu.async_copy(ht_hbm, tbl_v.at[pl.ds(MT_WORDS, HT_WORDS)], sem),
        pltpu.async_copy(months_hbm.at[pl.ds(base, BPW)], m_v, sem),
        pltpu.async_copy(hours_hbm.at[pl.ds(base, BPW)], h_v, sem),
    ]
    for c in copies:
        c.wait()

    lane = lax.iota(jnp.int32, L)
    rotv = [(lane + r) & (L - 1) for r in range(L)]
    posv = [lane * D + ((lane + r) & (L - 1)) for r in range(L)]

    PIPE = 12

    def build(g):
        mm = m_v[pl.ds(g * L, L)]
        hh = h_v[pl.ds(g * L, L)]
        mb = mm * HALF
        hb = hh * HALF
        gbase = g * (L * D)
        addr = []
        for r in range(L):
            am = mb + rotv[r]
            ah = hb + rotv[r]
            pv = posv[r]
            for q in range(0, HALF, L):
                addr.append((am + q, pv + (gbase + q)))
                addr.append((ah + (MT_WORDS + q), pv + (gbase + HALF + q)))
        pend = []
        for idx, pos in addr:
            pend.append((pos, plsc.load_gather(tbl_v, [idx])))
            if len(pend) > PIPE:
                p, v = pend.pop(0)
                plsc.store_scatter(rows_v, [p], v)
        for p, v in pend:
            plsc.store_scatter(rows_v, [p], v)

    HALF_WORDS = BPW * D // 2

    @pl.loop(0, GROUPS // 2)
    def _(g):
        build(g)

    wcopy = pltpu.async_copy(
        rows_v.at[pl.ds(0, HALF_WORDS)],
        out_hbm.at[pl.ds(base * D, HALF_WORDS)], sem)

    @pl.loop(GROUPS // 2, GROUPS)
    def _(g):
        build(g)

    wcopy.wait()
    pltpu.sync_copy(
        rows_v.at[pl.ds(HALF_WORDS, HALF_WORDS)],
        out_hbm.at[pl.ds(base * D + HALF_WORDS, HALF_WORDS)])


def kernel(months, hours, month_table, hour_table):
    mesh = plsc.VectorSubcoreMesh(core_axis_name="c", subcore_axis_name="s")
    cp = pltpu.CompilerParams(needs_layout_passes=False, use_tc_tiling_on_sc=False)
    run = pl.kernel(
        _emb_body,
        out_type=jax.ShapeDtypeStruct((B * D,), jnp.float32),
        mesh=mesh,
        scratch_types=[
            pltpu.VMEM((TBL_WORDS,), jnp.float32),
            pltpu.VMEM((BPW,), jnp.int32),
            pltpu.VMEM((BPW,), jnp.int32),
            pltpu.VMEM((BPW * D,), jnp.float32),
            pltpu.SemaphoreType.DMA,
        ],
        compiler_params=cp,
    )
    out = run(month_table.reshape(-1), hour_table.reshape(-1),
              months.astype(jnp.int32), hours.astype(jnp.int32))
    return out.reshape(B, D)


# speedup vs baseline: 4.9254x; 1.2713x over previous
import jax
import jax.numpy as jnp
from jax import lax
from jax.experimental import pallas as pl
from jax.experimental.pallas import tpu as pltpu
from jax.experimental.pallas import tpu_sc as plsc

B = 16384
D = 128
HALF = 64
NC = 2
NS = 16
L = 16
NW = NC * NS
BPW = B // NW
GROUPS = BPW // L
MT_WORDS = 12 * HALF
HT_WORDS = 24 * HALF
TBL_WORDS = MT_WORDS + HT_WORDS


def _emb_body(mt_hbm, ht_hbm, months_hbm, hours_hbm, out_hbm,
              tbl_v, m_v, h_v, rows_v, sem):
    wid = lax.axis_index("s") * NC + lax.axis_index("c")
    base = wid * BPW
    copies = [
        pltpu.async_copy(mt_hbm, tbl_v.at[pl.ds(0, MT_WORDS)], sem),
        pltpu.async_copy(ht_hbm, tbl_v.at[pl.ds(MT_WORDS, HT_WORDS)], sem),
        pltpu.async_copy(months_hbm.at[pl.ds(base, BPW)], m_v, sem),
        pltpu.async_copy(hours_hbm.at[pl.ds(base, BPW)], h_v, sem),
    ]
    for c in copies:
        c.wait()

    lane = lax.iota(jnp.int32, L)
    lanec = [lane + c for c in range(0, HALF, L)]
    spl = [jnp.full((L,), l, jnp.int32) for l in range(L)]

    PIPE = 12

    def build(g):
        mm = m_v[pl.ds(g * L, L)]
        hh = h_v[pl.ds(g * L, L)]
        mb = mm * HALF
        hb = hh * HALF + MT_WORDS
        gbase = g * (L * D)
        pend = []

        def drain():
            off, v = pend.pop(0)
            rows_v[pl.ds(off, L)] = v

        for l in range(L):
            bm = mb.at[spl[l]].get(mode="promise_in_bounds")
            bh = hb.at[spl[l]].get(mode="promise_in_bounds")
            for ci, c in enumerate(range(0, HALF, L)):
                pend.append((gbase + l * D + c,
                             plsc.load_gather(tbl_v, [bm + lanec[ci]])))
                pend.append((gbase + l * D + HALF + c,
                             plsc.load_gather(tbl_v, [bh + lanec[ci]])))
                while len(pend) > PIPE:
                    drain()
        while pend:
            drain()

    NCHUNK = 4
    CG = GROUPS // NCHUNK
    CW = BPW * D // NCHUNK

    def _chunk_copy(c):
        return pltpu.make_async_copy(
            rows_v.at[pl.ds(c * CW, CW)],
            out_hbm.at[pl.ds(base * D + c * CW, CW)], sem)

    @pl.loop(0, GROUPS)
    def _(g):
        build(g)
        for c in range(NCHUNK - 1):
            @pl.when(g == (c + 1) * CG - 1)
            def _(c=c):
                _chunk_copy(c).start()

    pltpu.sync_copy(
        rows_v.at[pl.ds((NCHUNK - 1) * CW, CW)],
        out_hbm.at[pl.ds(base * D + (NCHUNK - 1) * CW, CW)])
    for c in range(NCHUNK - 1):
        _chunk_copy(c).wait()


def kernel(months, hours, month_table, hour_table):
    mesh = plsc.VectorSubcoreMesh(core_axis_name="c", subcore_axis_name="s")
    cp = pltpu.CompilerParams(needs_layout_passes=False, use_tc_tiling_on_sc=False)
    run = pl.kernel(
        _emb_body,
        out_type=jax.ShapeDtypeStruct((B * D,), jnp.float32),
        mesh=mesh,
        scratch_types=[
            pltpu.VMEM((TBL_WORDS,), jnp.float32),
            pltpu.VMEM((BPW,), jnp.int32),
            pltpu.VMEM((BPW,), jnp.int32),
            pltpu.VMEM((BPW * D,), jnp.float32),
            pltpu.SemaphoreType.DMA,
        ],
        compiler_params=cp,
    )
    out = run(month_table.reshape(-1), hour_table.reshape(-1),
              months.astype(jnp.int32), hours.astype(jnp.int32))
    return out.reshape(B, D)


# speedup vs baseline: 4.9442x; 1.0038x over previous
import jax
import jax.numpy as jnp
from jax import lax
from jax.experimental import pallas as pl
from jax.experimental.pallas import tpu as pltpu
from jax.experimental.pallas import tpu_sc as plsc

B = 16384
D = 128
HALF = 64
NC = 2
NS = 16
L = 16
NW = NC * NS
BPW = B // NW
GROUPS = BPW // L
MT_WORDS = 12 * HALF
HT_WORDS = 24 * HALF
TBL_WORDS = MT_WORDS + HT_WORDS


def _emb_body(mt_hbm, ht_hbm, months_hbm, hours_hbm, out_hbm,
              tbl_v, m_v, h_v, rows_v, sem):
    wid = lax.axis_index("s") * NC + lax.axis_index("c")
    base = wid * BPW
    copies = [
        pltpu.async_copy(mt_hbm, tbl_v.at[pl.ds(0, MT_WORDS)], sem),
        pltpu.async_copy(ht_hbm, tbl_v.at[pl.ds(MT_WORDS, HT_WORDS)], sem),
        pltpu.async_copy(months_hbm.at[pl.ds(base, BPW)], m_v, sem),
        pltpu.async_copy(hours_hbm.at[pl.ds(base, BPW)], h_v, sem),
    ]
    for c in copies:
        c.wait()

    lane = lax.iota(jnp.int32, L)
    lanec = [lane + c for c in range(0, HALF, L)]
    spl = [jnp.full((L,), l, jnp.int32) for l in range(L)]

    PIPE = 12

    def build(g):
        mm = m_v[pl.ds(g * L, L)]
        hh = h_v[pl.ds(g * L, L)]
        mb = mm * HALF
        hb = hh * HALF + MT_WORDS
        gbase = g * (L * D)
        pend = []

        def drain():
            off, v = pend.pop(0)
            rows_v[pl.ds(off, L)] = v

        for l in range(L):
            bm = mb.at[spl[l]].get(mode="promise_in_bounds")
            bh = hb.at[spl[l]].get(mode="promise_in_bounds")
            for ci, c in enumerate(range(0, HALF, L)):
                pend.append((gbase + l * D + c,
                             plsc.load_gather(tbl_v, [bm + lanec[ci]])))
                pend.append((gbase + l * D + HALF + c,
                             plsc.load_gather(tbl_v, [bh + lanec[ci]])))
                while len(pend) > PIPE:
                    drain()
        while pend:
            drain()

    NCHUNK = 8
    CG = GROUPS // NCHUNK
    CW = BPW * D // NCHUNK

    def _chunk_copy(c):
        return pltpu.make_async_copy(
            rows_v.at[pl.ds(c * CW, CW)],
            out_hbm.at[pl.ds(base * D + c * CW, CW)], sem)

    @pl.loop(0, GROUPS)
    def _(g):
        build(g)
        for c in range(NCHUNK - 1):
            @pl.when(g == (c + 1) * CG - 1)
            def _(c=c):
                _chunk_copy(c).start()

    pltpu.sync_copy(
        rows_v.at[pl.ds((NCHUNK - 1) * CW, CW)],
        out_hbm.at[pl.ds(base * D + (NCHUNK - 1) * CW, CW)])
    for c in range(NCHUNK - 1):
        _chunk_copy(c).wait()


def kernel(months, hours, month_table, hour_table):
    mesh = plsc.VectorSubcoreMesh(core_axis_name="c", subcore_axis_name="s")
    cp = pltpu.CompilerParams(needs_layout_passes=False, use_tc_tiling_on_sc=False)
    run = pl.kernel(
        _emb_body,
        out_type=jax.ShapeDtypeStruct((B * D,), jnp.float32),
        mesh=mesh,
        scratch_types=[
            pltpu.VMEM((TBL_WORDS,), jnp.float32),
            pltpu.VMEM((BPW,), jnp.int32),
            pltpu.VMEM((BPW,), jnp.int32),
            pltpu.VMEM((BPW * D,), jnp.float32),
            pltpu.SemaphoreType.DMA,
        ],
        compiler_params=cp,
    )
    out = run(month_table.reshape(-1), hour_table.reshape(-1),
              months.astype(jnp.int32), hours.astype(jnp.int32))
    return out.reshape(B, D)


# speedup vs baseline: 4.9468x; 1.0005x over previous
import jax
import jax.numpy as jnp
from jax import lax
from jax.experimental import pallas as pl
from jax.experimental.pallas import tpu as pltpu
from jax.experimental.pallas import tpu_sc as plsc

B = 16384
D = 128
HALF = 64
NC = 2
NS = 16
L = 16
NW = NC * NS
BPW = B // NW
GROUPS = BPW // L
MT_WORDS = 12 * HALF
HT_WORDS = 24 * HALF
TBL_WORDS = MT_WORDS + HT_WORDS


def _emb_body(mt_hbm, ht_hbm, months_hbm, hours_hbm, out_hbm,
              tbl_v, m_v, h_v, rows_v, sem):
    wid = lax.axis_index("s") * NC + lax.axis_index("c")
    base = wid * BPW
    copies = [
        pltpu.async_copy(mt_hbm, tbl_v.at[pl.ds(0, MT_WORDS)], sem),
        pltpu.async_copy(ht_hbm, tbl_v.at[pl.ds(MT_WORDS, HT_WORDS)], sem),
        pltpu.async_copy(months_hbm.at[pl.ds(base, BPW)], m_v, sem),
        pltpu.async_copy(hours_hbm.at[pl.ds(base, BPW)], h_v, sem),
    ]
    for c in copies:
        c.wait()

    lane = lax.iota(jnp.int32, L)
    lanec = [lane + c for c in range(0, HALF, L)]
    spl = [jnp.full((L,), l, jnp.int32) for l in range(L)]

    PIPE = 12

    def build(g):
        mm = m_v[pl.ds(g * L, L)]
        hh = h_v[pl.ds(g * L, L)]
        mb = mm * HALF
        hb = hh * HALF + MT_WORDS
        gbase = g * (L * D)
        pend = []

        def drain():
            off, v = pend.pop(0)
            rows_v[pl.ds(off, L)] = v

        for l in range(L):
            bm = mb.at[spl[l]].get(mode="promise_in_bounds")
            bh = hb.at[spl[l]].get(mode="promise_in_bounds")
            for ci, c in enumerate(range(0, HALF, L)):
                pend.append((gbase + l * D + c,
                             plsc.load_gather(tbl_v, [bm + lanec[ci]])))
                pend.append((gbase + l * D + HALF + c,
                             plsc.load_gather(tbl_v, [bh + lanec[ci]])))
                while len(pend) > PIPE:
                    drain()
        while pend:
            drain()

    NCHUNK = 8
    CG = GROUPS // NCHUNK
    CW = BPW * D // NCHUNK

    def _chunk_copy(c):
        return pltpu.make_async_copy(
            rows_v.at[pl.ds(c * CW, CW)],
            out_hbm.at[pl.ds(base * D + c * CW, CW)], sem)

    @pl.loop(0, GROUPS)
    def _(g):
        build(g)
        for c in range(NCHUNK - 1):
            @pl.when(g == (c + 1) * CG - 1)
            def _(c=c):
                _chunk_copy(c).start()

    pltpu.sync_copy(
        rows_v.at[pl.ds((NCHUNK - 1) * CW, CW)],
        out_hbm.at[pl.ds(base * D + (NCHUNK - 1) * CW, CW)])
    for c in range(NCHUNK - 1):
        _chunk_copy(c).wait()


def kernel(months, hours, month_table, hour_table):
    mesh = plsc.VectorSubcoreMesh(core_axis_name="c", subcore_axis_name="s")
    cp = pltpu.CompilerParams(needs_layout_passes=False, use_tc_tiling_on_sc=False,
                              disable_bounds_checks=True,
                              disable_semaphore_checks=True)
    run = pl.kernel(
        _emb_body,
        out_type=jax.ShapeDtypeStruct((B * D,), jnp.float32),
        mesh=mesh,
        scratch_types=[
            pltpu.VMEM((TBL_WORDS,), jnp.float32),
            pltpu.VMEM((BPW,), jnp.int32),
            pltpu.VMEM((BPW,), jnp.int32),
            pltpu.VMEM((BPW * D,), jnp.float32),
            pltpu.SemaphoreType.DMA,
        ],
        compiler_params=cp,
    )
    out = run(month_table.reshape(-1), hour_table.reshape(-1),
              months.astype(jnp.int32), hours.astype(jnp.int32))
    return out.reshape(B, D)
